# baseline (speedup 1.0000x reference)
import functools

import jax
import jax.numpy as jnp
from jax import lax
from jax.experimental import pallas as pl
from jax.experimental.pallas import tpu as pltpu
from jax.experimental.pallas import tpu_sc as plsc

VOCAB = 1000000
EMBED = 64
NUM_CLASSES = 1000
BATCH = 16384

_info = plsc.get_sparse_core_info()
_NC, _NS = _info.num_cores, _info.num_subcores
_NW = _NC * _NS
_BPW = BATCH // _NW
_CHUNK = 128
_NCHUNK = _BPW // _CHUNK

_sc_mesh = plsc.VectorSubcoreMesh(core_axis_name="c", subcore_axis_name="s")


@functools.partial(
    pl.kernel,
    mesh=_sc_mesh,
    out_type=jax.ShapeDtypeStruct((BATCH, 2 * EMBED), jnp.float32),
    scratch_types=[
        pltpu.VMEM((_NCHUNK, _CHUNK), jnp.int32),
        pltpu.VMEM((_BPW, 2 * EMBED), jnp.float32),
        pltpu.SemaphoreType.DMA,
    ],
)
def _gather_sc(idx_hbm, table_hbm, out_hbm, idx_v, rows_v, sem):
    wid = lax.axis_index("s") * _NC + lax.axis_index("c")
    base = wid * _BPW
    pltpu.sync_copy(idx_hbm.at[wid], idx_v)
    copies = []
    for j in range(_NCHUNK):
        copies.append(
            pltpu.async_copy(
                table_hbm.at[idx_v.at[j]],
                rows_v.at[pl.ds(j * _CHUNK, _CHUNK)],
                sem,
            )
        )
    for c in copies:
        c.wait()
    pltpu.sync_copy(rows_v, out_hbm.at[pl.ds(base, _BPW)])


_BB = 1024


def _matmul_body(e2_ref, par_ref, w_ref, b_ref, o_ref):
    e2 = e2_ref[...]
    par = par_ref[...]
    x = jnp.where(par == 1, e2[:, EMBED:], e2[:, :EMBED])
    o_ref[...] = (
        jnp.dot(x, w_ref[...], preferred_element_type=jnp.float32) + b_ref[...]
    )


def _matmul_tc(emb2, par, W, b2d):
    return pl.pallas_call(
        _matmul_body,
        grid=(BATCH // _BB,),
        in_specs=[
            pl.BlockSpec((_BB, 2 * EMBED), lambda i: (i, 0)),
            pl.BlockSpec((_BB, 1), lambda i: (i, 0)),
            pl.BlockSpec((EMBED, NUM_CLASSES), lambda i: (0, 0)),
            pl.BlockSpec((1, NUM_CLASSES), lambda i: (0, 0)),
        ],
        out_specs=pl.BlockSpec((_BB, NUM_CLASSES), lambda i: (i, 0)),
        out_shape=jax.ShapeDtypeStruct((BATCH, NUM_CLASSES), jnp.float32),
    )(emb2, par, W, b2d)


def kernel(inputs, table, W, b):
    idx = inputs.astype(jnp.int32)
    line = (idx >> 1).reshape(_NW, _NCHUNK, _CHUNK)
    t2 = table.reshape(VOCAB // 2, 2 * EMBED)
    emb2 = _gather_sc(line, t2)
    par = (idx & 1).reshape(BATCH, 1)
    return _matmul_tc(emb2, par, W, b.reshape(1, NUM_CLASSES))


# speedup vs baseline: 1.0101x; 1.0101x over previous
import functools

import jax
import jax.numpy as jnp
from jax import lax
from jax.experimental import pallas as pl
from jax.experimental.pallas import tpu as pltpu
from jax.experimental.pallas import tpu_sc as plsc

VOCAB = 1000000
EMBED = 64
NUM_CLASSES = 1000
BATCH = 16384

_info = plsc.get_sparse_core_info()
_NC, _NS = _info.num_cores, _info.num_subcores
_NW = _NC * _NS
_BPW = BATCH // _NW
_CHUNK = 128
_NCHUNK = _BPW // _CHUNK

_sc_mesh = plsc.VectorSubcoreMesh(core_axis_name="c", subcore_axis_name="s")


@functools.partial(
    pl.kernel,
    mesh=_sc_mesh,
    out_type=jax.ShapeDtypeStruct((BATCH, EMBED), jnp.float32),
    scratch_types=[
        pltpu.VMEM((_NCHUNK, _CHUNK), jnp.int32),
        pltpu.VMEM((_BPW, EMBED), jnp.float32),
        pltpu.SemaphoreType.DMA,
    ],
    compiler_params=pltpu.CompilerParams(use_tc_tiling_on_sc=False),
)
def _gather_sc(idx_hbm, table_hbm, out_hbm, idx_v, rows_v, sem):
    wid = lax.axis_index("s") * _NC + lax.axis_index("c")
    base = wid * _BPW
    pltpu.sync_copy(idx_hbm.at[wid], idx_v)
    copies = []
    for j in range(_NCHUNK):
        copies.append(
            pltpu.async_copy(
                table_hbm.at[idx_v.at[j]],
                rows_v.at[pl.ds(j * _CHUNK, _CHUNK)],
                sem,
            )
        )
    for c in copies:
        c.wait()
    pltpu.sync_copy(rows_v, out_hbm.at[pl.ds(base, _BPW)])


_BB = 1024


def _matmul_body(x_ref, w_ref, b_ref, o_ref):
    o_ref[...] = (
        jnp.dot(x_ref[...], w_ref[...], preferred_element_type=jnp.float32)
        + b_ref[...]
    )


def _matmul_tc(emb, W, b2d):
    return pl.pallas_call(
        _matmul_body,
        grid=(BATCH // _BB,),
        in_specs=[
            pl.BlockSpec((_BB, EMBED), lambda i: (i, 0)),
            pl.BlockSpec((EMBED, NUM_CLASSES), lambda i: (0, 0)),
            pl.BlockSpec((1, NUM_CLASSES), lambda i: (0, 0)),
        ],
        out_specs=pl.BlockSpec((_BB, NUM_CLASSES), lambda i: (i, 0)),
        out_shape=jax.ShapeDtypeStruct((BATCH, NUM_CLASSES), jnp.float32),
    )(emb, W, b2d)


def kernel(inputs, table, W, b):
    idx = inputs.astype(jnp.int32).reshape(_NW, _NCHUNK, _CHUNK)
    emb = _gather_sc(idx, table)
    return _matmul_tc(emb, W, b.reshape(1, NUM_CLASSES))


# speedup vs baseline: 2.1041x; 2.0830x over previous
import functools

import jax
import jax.numpy as jnp
from jax import lax
from jax.experimental import pallas as pl
from jax.experimental.pallas import tpu as pltpu
from jax.experimental.pallas import tpu_sc as plsc

VOCAB = 1000000
EMBED = 64
NUM_CLASSES = 1000
BATCH = 16384

_info = plsc.get_sparse_core_info()
_NC, _NS = _info.num_cores, _info.num_subcores
_NW = _NC * _NS
_BPW = BATCH // _NW
_CHUNK = 128
_NCHUNK = _BPW // _CHUNK

_VB = 16384
_HB = _VB // 2
_NVB = (VOCAB + _VB - 1) // _VB
_PACK_ROWS = _NVB * _HB


def _pack_body(tt_ref, o_ref):
    x = tt_ref[...]
    o_ref[...] = jnp.concatenate(
        [x[:, :_HB].T, x[:, _HB:].T], axis=1
    )


def _pack_tc(tt):
    return pl.pallas_call(
        _pack_body,
        grid=(_NVB,),
        in_specs=[pl.BlockSpec((EMBED, _VB), lambda i: (0, i))],
        out_specs=pl.BlockSpec((_HB, 2 * EMBED), lambda i: (i, 0)),
        out_shape=jax.ShapeDtypeStruct((_PACK_ROWS, 2 * EMBED), jnp.float32),
    )(tt)


_sc_mesh = plsc.VectorSubcoreMesh(core_axis_name="c", subcore_axis_name="s")


@functools.partial(
    pl.kernel,
    mesh=_sc_mesh,
    out_type=jax.ShapeDtypeStruct((BATCH, 2 * EMBED), jnp.float32),
    scratch_types=[
        pltpu.VMEM((_NCHUNK, _CHUNK), jnp.int32),
        pltpu.VMEM((_BPW, 2 * EMBED), jnp.float32),
        pltpu.SemaphoreType.DMA,
    ],
)
def _gather_sc(idx_hbm, t2_hbm, out_hbm, idx_v, rows_v, sem):
    wid = lax.axis_index("s") * _NC + lax.axis_index("c")
    base = wid * _BPW
    pltpu.sync_copy(idx_hbm.at[wid], idx_v)
    copies = []
    for j in range(_NCHUNK):
        copies.append(
            pltpu.async_copy(
                t2_hbm.at[idx_v.at[j]],
                rows_v.at[pl.ds(j * _CHUNK, _CHUNK)],
                sem,
            )
        )
    for c in copies:
        c.wait()
    pltpu.sync_copy(rows_v, out_hbm.at[pl.ds(base, _BPW)])


_BB = 1024


def _matmul_body(e2_ref, par_ref, w_ref, b_ref, o_ref):
    e2 = e2_ref[...]
    par = par_ref[...]
    x = jnp.where(par == 1, e2[:, EMBED:], e2[:, :EMBED]).astype(jnp.bfloat16)
    o_ref[...] = (
        jnp.dot(x, w_ref[...], preferred_element_type=jnp.float32)
        + b_ref[...]
    )


def _matmul_tc(emb2, par, Wb, b2d):
    return pl.pallas_call(
        _matmul_body,
        grid=(BATCH // _BB,),
        in_specs=[
            pl.BlockSpec((_BB, 2 * EMBED), lambda i: (i, 0)),
            pl.BlockSpec((_BB, 1), lambda i: (i, 0)),
            pl.BlockSpec((EMBED, NUM_CLASSES), lambda i: (0, 0)),
            pl.BlockSpec((1, NUM_CLASSES), lambda i: (0, 0)),
        ],
        out_specs=pl.BlockSpec((_BB, NUM_CLASSES), lambda i: (i, 0)),
        out_shape=jax.ShapeDtypeStruct((BATCH, NUM_CLASSES), jnp.float32),
    )(emb2, par, Wb, b2d)


def kernel(inputs, table, W, b):
    idx = inputs.astype(jnp.int32)
    t2 = _pack_tc(table.T)
    blk = idx // _VB
    off = idx % _VB
    line = (blk * _HB + off % _HB).reshape(_NW, _NCHUNK, _CHUNK)
    emb2 = _gather_sc(line, t2)
    par = (off >= _HB).astype(jnp.int32).reshape(BATCH, 1)
    return _matmul_tc(
        emb2, par, W.astype(jnp.bfloat16), b.reshape(1, NUM_CLASSES)
    )


# speedup vs baseline: 2.2290x; 1.0593x over previous
import functools

import jax
import jax.numpy as jnp
from jax import lax
from jax.experimental import pallas as pl
from jax.experimental.pallas import tpu as pltpu
from jax.experimental.pallas import tpu_sc as plsc

VOCAB = 1000000
EMBED = 64
NUM_CLASSES = 1000
BATCH = 16384

_info = plsc.get_sparse_core_info()
_NC, _NS = _info.num_cores, _info.num_subcores
_NW = _NC * _NS
_BPW = BATCH // _NW
_CHUNK = 128
_NCHUNK = _BPW // _CHUNK

_VB = 32768
_HB = _VB // 2
_NVB = (VOCAB + _VB - 1) // _VB
_PACK_ROWS = _NVB * _HB


def _pack_body(tt_ref, o_ref):
    x = tt_ref[...]
    o_ref[...] = jnp.concatenate([x[:, :_HB].T, x[:, _HB:].T], axis=1)


def _pack_tc(tt):
    return pl.pallas_call(
        _pack_body,
        grid=(_NVB,),
        in_specs=[pl.BlockSpec((EMBED, _VB), lambda i: (0, i))],
        out_specs=pl.BlockSpec((_HB, 2 * EMBED), lambda i: (i, 0)),
        out_shape=jax.ShapeDtypeStruct((_PACK_ROWS, 2 * EMBED), jnp.float32),
    )(tt)


_sc_mesh = plsc.VectorSubcoreMesh(core_axis_name="c", subcore_axis_name="s")


@functools.partial(
    pl.kernel,
    mesh=_sc_mesh,
    out_type=jax.ShapeDtypeStruct((BATCH, 2 * EMBED), jnp.float32),
    scratch_types=[
        pltpu.VMEM((_NCHUNK, _CHUNK), jnp.int32),
        pltpu.VMEM((_BPW, 2 * EMBED), jnp.float32),
        pltpu.SemaphoreType.DMA,
    ],
)
def _gather_sc(idx_hbm, t2_hbm, out_hbm, idx_v, rows_v, sem):
    wid = lax.axis_index("s") * _NC + lax.axis_index("c")
    base = wid * _BPW
    pltpu.sync_copy(idx_hbm.at[wid], idx_v)
    copies = []
    for j in range(_NCHUNK):
        copies.append(
            pltpu.async_copy(
                t2_hbm.at[idx_v.at[j]],
                rows_v.at[pl.ds(j * _CHUNK, _CHUNK)],
                sem,
            )
        )
    for c in copies:
        c.wait()
    pltpu.sync_copy(rows_v, out_hbm.at[pl.ds(base, _BPW)])


_BB = 2048


def _matmul_body(e2_ref, par_ref, w_ref, b_ref, o_ref):
    e2 = e2_ref[...]
    par = par_ref[...]
    x = jnp.where(par == 1, e2[:, EMBED:], e2[:, :EMBED]).astype(jnp.bfloat16)
    o_ref[...] = (
        jnp.dot(x, w_ref[...], preferred_element_type=jnp.float32)
        + b_ref[...]
    )


def _matmul_tc(emb2, par, Wb, b2d):
    return pl.pallas_call(
        _matmul_body,
        grid=(BATCH // _BB,),
        in_specs=[
            pl.BlockSpec((_BB, 2 * EMBED), lambda i: (i, 0)),
            pl.BlockSpec((_BB, 1), lambda i: (i, 0)),
            pl.BlockSpec((EMBED, NUM_CLASSES), lambda i: (0, 0)),
            pl.BlockSpec((1, NUM_CLASSES), lambda i: (0, 0)),
        ],
        out_specs=pl.BlockSpec((_BB, NUM_CLASSES), lambda i: (i, 0)),
        out_shape=jax.ShapeDtypeStruct((BATCH, NUM_CLASSES), jnp.float32),
    )(emb2, par, Wb, b2d)


def kernel(inputs, table, W, b):
    idx = inputs.astype(jnp.int32)
    t2 = _pack_tc(table.T)
    blk = idx // _VB
    off = idx % _VB
    line = (blk * _HB + off % _HB).reshape(_NW, _NCHUNK, _CHUNK)
    emb2 = _gather_sc(line, t2)
    par = (off >= _HB).astype(jnp.int32).reshape(BATCH, 1)
    return _matmul_tc(
        emb2, par, W.astype(jnp.bfloat16), b.reshape(1, NUM_CLASSES)
    )
